# Initial kernel scaffold; baseline (speedup 1.0000x reference)
import functools

import jax
import jax.numpy as jnp
from jax import lax
from jax.experimental import pallas as pl
from jax.experimental.pallas import tpu as pltpu
from jax.experimental.pallas import tpu_sc as plsc

N = 10000
E = 320000
D = 128
NW = 32
CPT = 79
EPT = CPT * 128
E_PAD = NW * EPT
NPAD = 10240
RPT = NPAD // 16

_mesh = plsc.VectorSubcoreMesh(core_axis_name="c", subcore_axis_name="s")


@functools.partial(
    pl.kernel,
    mesh=_mesh,
    out_type=(
        jax.ShapeDtypeStruct((2, NPAD, D), jnp.float32),
        jax.ShapeDtypeStruct((2, NPAD, 16), jnp.float32),
    ),
    scratch_types=[
        pltpu.VMEM((CPT, 128), jnp.int32),
        pltpu.VMEM((CPT, 128), jnp.int32),
        pltpu.VMEM((128, D), jnp.float32),
        pltpu.VMEM((128, 16), jnp.float32),
        pltpu.VMEM((128, 16), jnp.float32),
        pltpu.VMEM_SHARED((NPAD, D), jnp.float32),
        pltpu.VMEM_SHARED((NPAD, 16), jnp.float32),
        pltpu.SemaphoreType.DMA,
    ],
)
def _sc_agg(h_hbm, src_hbm, dst_hbm, agg_out, deg_out,
            src_v, dst_v, rows_v, ones_v, z16_v, agg_sh, deg_sh, sem):
    cid = lax.axis_index("c")
    tid = lax.axis_index("s")
    wid = cid * 16 + tid

    pltpu.sync_copy(src_hbm.at[wid], src_v)
    pltpu.sync_copy(dst_hbm.at[wid], dst_v)

    @pl.loop(0, 128)
    def _(r):
        for c8 in range(D // 16):
            rows_v[r, pl.ds(c8 * 16, 16)] = jnp.zeros((16,), jnp.float32)
        ones_v[r, :] = jnp.ones((16,), jnp.float32)
        z16_v[r, :] = jnp.zeros((16,), jnp.float32)

    for kb in range(RPT // 128):
        off = tid * RPT + kb * 128
        pltpu.sync_copy(rows_v, agg_sh.at[pl.ds(off, 128)])
        pltpu.sync_copy(z16_v, deg_sh.at[pl.ds(off, 128)])
    plsc.subcore_barrier()

    @pl.loop(0, CPT)
    def _(g):
        pltpu.async_copy(h_hbm.at[src_v.at[g]], rows_v, sem).wait()
        pltpu.sync_copy(rows_v, agg_sh.at[dst_v.at[g]], add=True)
        pltpu.sync_copy(ones_v, deg_sh.at[dst_v.at[g]], add=True)

    plsc.subcore_barrier()

    for kb in range(RPT // 128):
        off = tid * RPT + kb * 128
        pltpu.sync_copy(agg_sh.at[pl.ds(off, 128)], agg_out.at[cid, pl.ds(off, 128)])
        pltpu.sync_copy(deg_sh.at[pl.ds(off, 128)], deg_out.at[cid, pl.ds(off, 128)])


def _tc_body(h_ref, a0_ref, a1_ref, d0_ref, d1_ref, ws_ref, wn_ref, b_ref, o_ref):
    deg = jnp.maximum(d0_ref[:, 0:1] + d1_ref[:, 0:1], 1.0)
    hn = (a0_ref[...] + a1_ref[...]) / deg
    dn = (((1,), (0,)), ((), ()))
    acc = lax.dot_general(h_ref[...], ws_ref[...], dimension_numbers=dn,
                          precision=lax.Precision.HIGHEST,
                          preferred_element_type=jnp.float32)
    acc = acc + lax.dot_general(hn, wn_ref[...], dimension_numbers=dn,
                                precision=lax.Precision.HIGHEST,
                                preferred_element_type=jnp.float32)
    o_ref[...] = jnp.maximum(acc + b_ref[...], 0.0)


_BLK = 400

_tc_layer = pl.pallas_call(
    _tc_body,
    grid=(N // _BLK,),
    in_specs=[
        pl.BlockSpec((_BLK, D), lambda i: (i, 0)),
        pl.BlockSpec((_BLK, D), lambda i: (i, 0)),
        pl.BlockSpec((_BLK, D), lambda i: (i, 0)),
        pl.BlockSpec((_BLK, 16), lambda i: (i, 0)),
        pl.BlockSpec((_BLK, 16), lambda i: (i, 0)),
        pl.BlockSpec((D, D), lambda i: (0, 0)),
        pl.BlockSpec((D, D), lambda i: (0, 0)),
        pl.BlockSpec((1, D), lambda i: (0, 0)),
    ],
    out_specs=pl.BlockSpec((_BLK, D), lambda i: (i, 0)),
    out_shape=jax.ShapeDtypeStruct((N, D), jnp.float32),
)


def kernel(x, edge_index, W_self_0, W_neigh_0, b_0, W_self_1, W_neigh_1, b_1,
           W_self_2, W_neigh_2, b_2):
    src = edge_index[0]
    dst = edge_index[1]
    pad = E_PAD - E
    src3 = jnp.concatenate([src, jnp.zeros((pad,), jnp.int32)]).reshape(NW, CPT, 128)
    dst3 = jnp.concatenate([dst, jnp.full((pad,), N, jnp.int32)]).reshape(NW, CPT, 128)

    params = [(W_self_0, W_neigh_0, b_0), (W_self_1, W_neigh_1, b_1),
              (W_self_2, W_neigh_2, b_2)]
    h = x
    for (Ws, Wn, b) in params:
        agg2, deg2 = _sc_agg(h, src3, dst3)
        h = _tc_layer(h, agg2[0, :N], agg2[1, :N], deg2[0, :N], deg2[1, :N],
                      Ws, Wn, b.reshape(1, D))
    return h


# speedup vs baseline: 2.0030x; 2.0030x over previous
import dataclasses
import functools

import jax
import jax.numpy as jnp
from jax import lax
from jax.experimental import pallas as pl
from jax.experimental.pallas import tpu as pltpu
from jax.experimental.pallas import tpu_sc as plsc

N = 10000
E = 320000
D = 128
NT = 16
CPT = 160
GRP = 8
E_PAD = NT * CPT * 128
PAD_DST = 1 << 21
NH = 5120
ACC = NH + 128
RPT = NH // NT
SEGS = (128, 128, 64)

_mesh = plsc.VectorSubcoreMesh(core_axis_name="c", subcore_axis_name="s")

_cp = pltpu.CompilerParams()
if "needs_layout_passes" in pltpu.CompilerParams.__dataclass_fields__:
    _cp = dataclasses.replace(_cp, needs_layout_passes=False)


@functools.partial(
    pl.kernel,
    mesh=_mesh,
    compiler_params=_cp,
    out_type=jax.ShapeDtypeStruct((2 * NH, D), jnp.float32),
    scratch_types=[
        pltpu.VMEM((GRP, 128), jnp.int32),
        pltpu.VMEM((GRP, 128), jnp.int32),
        pltpu.VMEM((128,), jnp.int32),
        pltpu.VMEM((128, D), jnp.float32),
        pltpu.VMEM((ACC,), jnp.float32),
        pltpu.VMEM((NT * 384,), jnp.float32),
        pltpu.VMEM((RPT,), jnp.float32),
        pltpu.VMEM_SHARED((ACC, D), jnp.float32),
        pltpu.VMEM_SHARED((NT * ACC,), jnp.float32),
        pltpu.SemaphoreType.DMA,
    ],
)
def _sc_mean(h_hbm, src_hbm, dst_hbm, out_hbm,
             src_v, dst_v, dstl_v, rows_v, degt_v, red_v, degr_v,
             agg_sh, stage_sh, sem):
    cid = lax.axis_index("c")
    tid = lax.axis_index("s")
    base = tid * RPT
    lo = cid * NH

    @pl.loop(0, 128)
    def _(r):
        for c8 in range(D // 16):
            rows_v[r, pl.ds(c8 * 16, 16)] = jnp.zeros((16,), jnp.float32)

    @pl.loop(0, ACC // 16)
    def _(r):
        degt_v[pl.ds(r * 16, 16)] = jnp.zeros((16,), jnp.float32)

    off = 0
    for seg in SEGS:
        pltpu.sync_copy(rows_v.at[pl.ds(0, seg)], agg_sh.at[pl.ds(base + off, seg)])
        off += seg
    plsc.subcore_barrier()

    ones16 = jnp.ones((16,), jnp.float32)

    @pl.loop(0, CPT // GRP)
    def _(go):
        pltpu.sync_copy(src_hbm.at[tid, pl.ds(go * GRP, GRP)], src_v)
        pltpu.sync_copy(dst_hbm.at[tid, pl.ds(go * GRP, GRP)], dst_v)

        @pl.loop(0, GRP)
        def _(g):
            cp = pltpu.async_copy(h_hbm.at[src_v.at[g]], rows_v, sem)
            for k in range(128 // 16):
                t = dst_v[g, pl.ds(k * 16, 16)] - lo
                valid = (t >= 0) & (t < NH)
                tl = jnp.where(valid, t, NH)
                dstl_v[pl.ds(k * 16, 16)] = tl
                plsc.addupdate_scatter(degt_v, [tl], ones16)
            cp.wait()
            pltpu.sync_copy(rows_v, agg_sh.at[dstl_v], add=True)

    pltpu.sync_copy(degt_v, stage_sh.at[pl.ds(tid * ACC, ACC)])
    plsc.subcore_barrier()

    loc = lax.rem(base, 128)
    wstart = base - loc
    for tt in range(NT):
        pltpu.sync_copy(stage_sh.at[pl.ds(tt * ACC + wstart, 384)],
                        red_v.at[pl.ds(tt * 384, 384)])

    @pl.loop(0, RPT // 16)
    def _(j):
        acc = red_v[pl.ds(loc + j * 16, 16)]
        for tt in range(1, NT):
            acc = acc + red_v[pl.ds(tt * 384 + loc + j * 16, 16)]
        degr_v[pl.ds(j * 16, 16)] = jnp.maximum(acc, 1.0)

    off = 0
    for seg in SEGS:
        pltpu.sync_copy(agg_sh.at[pl.ds(base + off, seg)], rows_v.at[pl.ds(0, seg)])

        @pl.loop(0, seg)
        def _(r):
            d = plsc.load_gather(degr_v, [jnp.full((16,), off + r, jnp.int32)])
            for c8 in range(D // 16):
                rows_v[r, pl.ds(c8 * 16, 16)] = rows_v[r, pl.ds(c8 * 16, 16)] / d

        pltpu.sync_copy(rows_v.at[pl.ds(0, seg)],
                        out_hbm.at[pl.ds(lo + base + off, seg)])
        off += seg


def _tc_body(h_ref, hn_ref, ws_ref, wn_ref, b_ref, o_ref):
    dn = (((1,), (0,)), ((), ()))
    acc = lax.dot_general(h_ref[...], ws_ref[...], dimension_numbers=dn,
                          precision=lax.Precision.HIGHEST,
                          preferred_element_type=jnp.float32)
    acc = acc + lax.dot_general(hn_ref[...], wn_ref[...], dimension_numbers=dn,
                                precision=lax.Precision.HIGHEST,
                                preferred_element_type=jnp.float32)
    o_ref[...] = jnp.maximum(acc + b_ref[...], 0.0)


_BLK = 400

_tc_layer = pl.pallas_call(
    _tc_body,
    grid=(N // _BLK,),
    in_specs=[
        pl.BlockSpec((_BLK, D), lambda i: (i, 0)),
        pl.BlockSpec((_BLK, D), lambda i: (i, 0)),
        pl.BlockSpec((D, D), lambda i: (0, 0)),
        pl.BlockSpec((D, D), lambda i: (0, 0)),
        pl.BlockSpec((1, D), lambda i: (0, 0)),
    ],
    out_specs=pl.BlockSpec((_BLK, D), lambda i: (i, 0)),
    out_shape=jax.ShapeDtypeStruct((N, D), jnp.float32),
)


def kernel(x, edge_index, W_self_0, W_neigh_0, b_0, W_self_1, W_neigh_1, b_1,
           W_self_2, W_neigh_2, b_2):
    src = edge_index[0]
    dst = edge_index[1]
    pad = E_PAD - E
    src3 = jnp.concatenate([src, jnp.zeros((pad,), jnp.int32)]).reshape(NT, CPT, 128)
    dst3 = jnp.concatenate([dst, jnp.full((pad,), PAD_DST, jnp.int32)]).reshape(NT, CPT, 128)

    params = [(W_self_0, W_neigh_0, b_0), (W_self_1, W_neigh_1, b_1),
              (W_self_2, W_neigh_2, b_2)]
    h = x
    for (Ws, Wn, b) in params:
        hn = _sc_mean(h, src3, dst3)
        h = _tc_layer(h, hn[:N], Ws, Wn, b.reshape(1, D))
    return h


# speedup vs baseline: 2.1084x; 1.0526x over previous
import dataclasses
import functools

import jax
import jax.numpy as jnp
from jax import lax
from jax.experimental import pallas as pl
from jax.experimental.pallas import tpu as pltpu
from jax.experimental.pallas import tpu_sc as plsc

N = 10000
E = 320000
D = 128
NT = 16
CPT = 160
GRPC = 16
NG = CPT // GRPC
E_PAD = NT * CPT * 128
PAD_DST = 1 << 21
NH = 5120
ACC = NH + 128
RPT = NH // NT
SEGS = (128, 128, 64)

_mesh = plsc.VectorSubcoreMesh(core_axis_name="c", subcore_axis_name="s")

_cp = pltpu.CompilerParams()
if "needs_layout_passes" in pltpu.CompilerParams.__dataclass_fields__:
    _cp = dataclasses.replace(_cp, needs_layout_passes=False)


@functools.partial(
    pl.kernel,
    mesh=_mesh,
    compiler_params=_cp,
    out_type=jax.ShapeDtypeStruct((2 * NH, D), jnp.float32),
    scratch_types=[
        pltpu.VMEM((GRPC, 128), jnp.int32),
        pltpu.VMEM((GRPC, 128), jnp.int32),
        pltpu.VMEM((128,), jnp.int32),
        pltpu.VMEM((128,), jnp.int32),
        pltpu.VMEM((128, D), jnp.float32),
        pltpu.VMEM((128, D), jnp.float32),
        pltpu.VMEM((ACC,), jnp.float32),
        pltpu.VMEM((NT * 384,), jnp.float32),
        pltpu.VMEM((RPT,), jnp.float32),
        pltpu.VMEM_SHARED((ACC, D), jnp.float32),
        pltpu.VMEM_SHARED((NT * ACC,), jnp.float32),
        pltpu.SemaphoreType.DMA,
        pltpu.SemaphoreType.DMA,
    ],
)
def _sc_mean(h_hbm, src_hbm, dst_hbm, out_hbm,
             src_v, dst_v, dl0_v, dl1_v, rows_v, rows1_v, degt_v, red_v, degr_v,
             agg_sh, stage_sh, sem0, sem1):
    cid = lax.axis_index("c")
    tid = lax.axis_index("s")
    base = tid * RPT
    lo = cid * NH

    @pl.loop(0, 128)
    def _(r):
        for c8 in range(D // 16):
            rows_v[r, pl.ds(c8 * 16, 16)] = jnp.zeros((16,), jnp.float32)

    @pl.loop(0, ACC // 16)
    def _(r):
        degt_v[pl.ds(r * 16, 16)] = jnp.zeros((16,), jnp.float32)

    off = 0
    for seg in SEGS:
        pltpu.sync_copy(rows_v.at[pl.ds(0, seg)], agg_sh.at[pl.ds(base + off, seg)])
        off += seg
    plsc.subcore_barrier()

    ones16 = jnp.ones((16,), jnp.float32)

    def _remap(g, dl):
        for k in range(128 // 16):
            t = dst_v[g, pl.ds(k * 16, 16)] - lo
            valid = (t >= 0) & (t < NH)
            tl = jnp.where(valid, t, NH)
            dl[pl.ds(k * 16, 16)] = tl
            plsc.addupdate_scatter(degt_v, [tl], ones16)

    def _wait(buf, s):
        pltpu.make_async_copy(h_hbm.at[src_v.at[0]], buf, s).wait()

    for go in range(NG):
        pltpu.sync_copy(src_hbm.at[tid, pl.ds(go * GRPC, GRPC)], src_v)
        pltpu.sync_copy(dst_hbm.at[tid, pl.ds(go * GRPC, GRPC)], dst_v)
        _remap(0, dl0_v)
        pltpu.async_copy(h_hbm.at[src_v.at[0]], rows_v, sem0)

        @pl.loop(0, GRPC // 2)
        def _(p):
            b = 2 * p + 1
            _remap(b, dl1_v)
            pltpu.async_copy(h_hbm.at[src_v.at[b]], rows1_v, sem1)
            _wait(rows_v, sem0)
            pltpu.sync_copy(rows_v, agg_sh.at[dl0_v], add=True)

            @pl.when(p < GRPC // 2 - 1)
            def _():
                _remap(2 * p + 2, dl0_v)
                pltpu.async_copy(h_hbm.at[src_v.at[2 * p + 2]], rows_v, sem0)

            _wait(rows1_v, sem1)
            pltpu.sync_copy(rows1_v, agg_sh.at[dl1_v], add=True)

    pltpu.sync_copy(degt_v, stage_sh.at[pl.ds(tid * ACC, ACC)])
    plsc.subcore_barrier()

    loc = lax.rem(base, 128)
    wstart = base - loc
    for tt in range(NT):
        pltpu.sync_copy(stage_sh.at[pl.ds(tt * ACC + wstart, 384)],
                        red_v.at[pl.ds(tt * 384, 384)])

    @pl.loop(0, RPT // 16)
    def _(j):
        acc = red_v[pl.ds(loc + j * 16, 16)]
        for tt in range(1, NT):
            acc = acc + red_v[pl.ds(tt * 384 + loc + j * 16, 16)]
        degr_v[pl.ds(j * 16, 16)] = jnp.maximum(acc, 1.0)

    off = 0
    for seg in SEGS:
        pltpu.sync_copy(agg_sh.at[pl.ds(base + off, seg)], rows_v.at[pl.ds(0, seg)])

        @pl.loop(0, seg)
        def _(r):
            d = plsc.load_gather(degr_v, [jnp.full((16,), off + r, jnp.int32)])
            for c8 in range(D // 16):
                rows_v[r, pl.ds(c8 * 16, 16)] = rows_v[r, pl.ds(c8 * 16, 16)] / d

        pltpu.sync_copy(rows_v.at[pl.ds(0, seg)],
                        out_hbm.at[pl.ds(lo + base + off, seg)])
        off += seg


def _tc_body(h_ref, hn_ref, ws_ref, wn_ref, b_ref, o_ref):
    dn = (((1,), (0,)), ((), ()))
    acc = lax.dot_general(h_ref[...], ws_ref[...], dimension_numbers=dn,
                          precision=lax.Precision.HIGHEST,
                          preferred_element_type=jnp.float32)
    acc = acc + lax.dot_general(hn_ref[...], wn_ref[...], dimension_numbers=dn,
                                precision=lax.Precision.HIGHEST,
                                preferred_element_type=jnp.float32)
    o_ref[...] = jnp.maximum(acc + b_ref[...], 0.0)


_BLK = 400

_tc_layer = pl.pallas_call(
    _tc_body,
    grid=(N // _BLK,),
    in_specs=[
        pl.BlockSpec((_BLK, D), lambda i: (i, 0)),
        pl.BlockSpec((_BLK, D), lambda i: (i, 0)),
        pl.BlockSpec((D, D), lambda i: (0, 0)),
        pl.BlockSpec((D, D), lambda i: (0, 0)),
        pl.BlockSpec((1, D), lambda i: (0, 0)),
    ],
    out_specs=pl.BlockSpec((_BLK, D), lambda i: (i, 0)),
    out_shape=jax.ShapeDtypeStruct((N, D), jnp.float32),
)


def kernel(x, edge_index, W_self_0, W_neigh_0, b_0, W_self_1, W_neigh_1, b_1,
           W_self_2, W_neigh_2, b_2):
    src = edge_index[0]
    dst = edge_index[1]
    pad = E_PAD - E
    src3 = jnp.concatenate([src, jnp.zeros((pad,), jnp.int32)]).reshape(NT, CPT, 128)
    dst3 = jnp.concatenate([dst, jnp.full((pad,), PAD_DST, jnp.int32)]).reshape(NT, CPT, 128)

    params = [(W_self_0, W_neigh_0, b_0), (W_self_1, W_neigh_1, b_1),
              (W_self_2, W_neigh_2, b_2)]
    h = x
    for (Ws, Wn, b) in params:
        hn = _sc_mean(h, src3, dst3)
        h = _tc_layer(h, hn[:N], Ws, Wn, b.reshape(1, D))
    return h
